# baseline (speedup 1.0000x reference)
import functools

import jax
import jax.numpy as jnp
from jax import lax
from jax.experimental import pallas as pl
from jax.experimental.pallas import tpu as pltpu
from jax.experimental.pallas import tpu_sc as plsc

NUM_EMBEDDINGS = 1000000
EMBEDDING_DIM = 64

NBATCH = 4096
NSEQ = 50
NUM_CORES = 2
NUM_SUBCORES = 16
NW = NUM_CORES * NUM_SUBCORES
COLS_PER_W = NBATCH // NW
NBUF = 5
NOUT = 2
INNER = 10
NGROUPS = NSEQ // INNER


def _gather_body(idx_hbm, table_hbm, out_hbm, idx_v, pcol_v, pair_v, out_v, *sems):
    gsems = sems[:NBUF]
    wsems = sems[NBUF:]
    wid = lax.axis_index("s") * NUM_CORES + lax.axis_index("c")
    col0 = wid * COLS_PER_W
    pltpu.sync_copy(idx_hbm.at[:, pl.ds(col0, COLS_PER_W)], idx_v)

    def prep(k, carry):
        s = k // 8
        c = (k % 8) * 16
        x = idx_v[s, pl.ds(c, 16)]
        idx_v[s, pl.ds(c, 16)] = lax.shift_right_logical(x, 1)
        pcol_v[s, pl.ds(c, 16)] = lax.shift_left(lax.bitwise_and(x, 1), 6)
        return carry

    lax.fori_loop(0, NSEQ * 8, prep, 0)

    def gather_start(s, b):
        pltpu.async_copy(table_hbm.at[idx_v.at[s]], pair_v.at[b], gsems[b])

    def gather_wait(b):
        pltpu.make_async_copy(
            table_hbm.at[idx_v.at[0]], pair_v.at[b], gsems[b]
        ).wait()

    def write_start(s, ob):
        pltpu.async_copy(
            out_v.at[ob], out_hbm.at[s, :, pl.ds(col0, COLS_PER_W)], wsems[ob]
        )

    def write_wait(ob):
        pltpu.make_async_copy(
            out_v.at[ob], out_hbm.at[0, :, pl.ds(col0, COLS_PER_W)], wsems[ob]
        ).wait()

    def transpose(s, b, ob):
        for l in range(COLS_PER_W // 16):
            tok = lax.iota(jnp.int32, 16) + (16 * l)
            pc = pcol_v[s, pl.ds(16 * l, 16)]

            def body(f, carry):
                x = plsc.load_gather(pair_v.at[b], [tok, pc + f])
                out_v[ob, f, pl.ds(16 * l, 16)] = x
                return carry

            lax.fori_loop(0, EMBEDDING_DIM, body, 0)

    for b in range(NBUF):
        gather_start(b, b)

    def group(g, carry):
        for k in range(INNER):
            s = g * INNER + k
            b = k % NBUF
            ob = k % NOUT
            gather_wait(b)

            @pl.when(s >= NOUT)
            def _():
                write_wait(ob)

            transpose(s, b, ob)

            @pl.when(s < NSEQ - NBUF)
            def _():
                gather_start(s + NBUF, b)

            write_start(s, ob)
        return carry

    lax.fori_loop(0, NGROUPS, group, 0)
    for ob in range(NOUT):
        write_wait(ob)


def kernel(input, weight):
    idx_t = input.T
    pair_table = weight.reshape(NUM_EMBEDDINGS // 2, 2 * EMBEDDING_DIM)
    mesh = plsc.VectorSubcoreMesh(core_axis_name="c", subcore_axis_name="s")
    run = functools.partial(
        pl.kernel,
        mesh=mesh,
        out_type=jax.ShapeDtypeStruct((NSEQ, EMBEDDING_DIM, NBATCH), jnp.float32),
        scratch_types=[
            pltpu.VMEM((NSEQ, COLS_PER_W), jnp.int32),
            pltpu.VMEM((NSEQ, COLS_PER_W), jnp.int32),
            pltpu.VMEM((NBUF, COLS_PER_W, 2 * EMBEDDING_DIM), jnp.float32),
            pltpu.VMEM((NOUT, EMBEDDING_DIM, COLS_PER_W), jnp.float32),
        ]
        + [pltpu.SemaphoreType.DMA] * (NBUF + NOUT),
        compiler_params=pltpu.CompilerParams(needs_layout_passes=False),
    )(_gather_body)
    out_t = run(idx_t, pair_table)
    return out_t.transpose(2, 0, 1)
